# baseline (speedup 1.0000x reference)
import functools

import jax
import jax.numpy as jnp
import numpy as np
from jax import lax
from jax.experimental import pallas as pl
from jax.experimental.pallas import tpu as pltpu
from jax.experimental.pallas import tpu_sc as plsc

BATCH = 4096
SEQ = 50
DIM = 128
LANES = 16
NCORE = 2
NSUB = 16
SENT_PER_W = BATCH // (NCORE * NSUB)
SENT_PER_SC = BATCH // NCORE
CHUNK_SENT = 2
CHUNK_ROWS = CHUNK_SENT * SEQ
NCHUNK = SENT_PER_W // CHUNK_SENT
NCHUNK_ALL = BATCH // CHUNK_SENT
LGROUPS = DIM // LANES
K_SC = 24
K_V = SEQ - K_SC
SC_ROWS = CHUNK_SENT * K_SC
NBUF = 4

_SEG = np.repeat((np.arange(BATCH, dtype=np.int32) % SENT_PER_SC)
                 .reshape(NCHUNK_ALL, CHUNK_SENT), K_SC, axis=1)


def _embed_body(idx_hbm, seg_hbm, table_hbm, out_hbm,
                idx_v, seg_v, rows_v, out_v, acc_v, acc_sp, gsems, ssems):
    c = lax.axis_index("c")
    s = lax.axis_index("s")
    sent_base = pl.multiple_of(c * (NSUB * SENT_PER_W) + s * SENT_PER_W,
                               SENT_PER_W)
    chunk_base = pl.multiple_of(sent_base // CHUNK_SENT, NCHUNK)
    slot_base = pl.multiple_of(s * SENT_PER_W, SENT_PER_W)

    pltpu.sync_copy(idx_hbm.at[pl.ds(chunk_base, NCHUNK)], idx_v)
    pltpu.sync_copy(seg_hbm.at[pl.ds(chunk_base, NCHUNK)], seg_v)

    zero = jnp.zeros((LANES,), jnp.float32)

    def zbody(r, carry):
        for l in range(LGROUPS):
            out_v[r, pl.ds(l * LANES, LANES)] = zero
        return carry

    lax.fori_loop(0, SENT_PER_W, zbody, 0)
    pltpu.sync_copy(out_v, acc_sp.at[pl.ds(slot_base, SENT_PER_W)])

    scale = jnp.float32(1.0 / SEQ)

    def gather(p, buf):
        return pltpu.async_copy(
            table_hbm.at[idx_v.at[p]], rows_v.at[buf], gsems.at[buf])

    def gather_wait(p, buf):
        pltpu.make_async_copy(
            table_hbm.at[idx_v.at[p]], rows_v.at[buf], gsems.at[buf]).wait()

    def scatter_add(p, buf):
        return pltpu.async_copy(
            rows_v.at[buf, pl.ds(0, SC_ROWS)],
            acc_sp.at[seg_v.at[p]], ssems.at[buf], add=True)

    def scatter_wait(p, buf):
        pltpu.make_async_copy(
            rows_v.at[buf, pl.ds(0, SC_ROWS)],
            acc_sp.at[seg_v.at[p]], ssems.at[buf]).wait()

    def reduce_sentence(buf, base_row):
        def body(t, accs):
            r0 = base_row + 2 * t
            return tuple(
                accs[l] + rows_v[buf, r0, pl.ds(l * LANES, LANES)]
                + rows_v[buf, r0 + 1, pl.ds(l * LANES, LANES)]
                for l in range(LGROUPS)
            )
        init = tuple(jnp.zeros((LANES,), jnp.float32) for _ in range(LGROUPS))
        return lax.fori_loop(0, K_V // 2, body, init)

    def compute(p, buf):
        for s2 in range(CHUNK_SENT):
            accs = reduce_sentence(buf, SC_ROWS + s2 * K_V)
            for l in range(LGROUPS):
                out_v[CHUNK_SENT * p + s2, pl.ds(l * LANES, LANES)] = accs[l]

    for j in range(NBUF):
        gather(j, j)

    def outer(q, carry):
        for j in range(NBUF):
            p = NBUF * q + j
            gather_wait(p, j)
            scatter_add(p, j)
            compute(p, j)
            scatter_wait(p, j)

            @pl.when(q < NCHUNK // NBUF - 1)
            def _():
                gather(p + NBUF, j)
        return carry

    lax.fori_loop(0, NCHUNK // NBUF, outer, 0)

    pltpu.sync_copy(acc_sp.at[pl.ds(slot_base, SENT_PER_W)], acc_v)

    def cbody(r, carry):
        for l in range(LGROUPS):
            sl = pl.ds(l * LANES, LANES)
            out_v[r, sl] = (out_v[r, sl] + acc_v[r, sl]) * scale
        return carry

    lax.fori_loop(0, SENT_PER_W, cbody, 0)

    pltpu.sync_copy(out_v, out_hbm.at[pl.ds(sent_base, SENT_PER_W)])


def _make():
    return functools.partial(
        pl.kernel,
        mesh=plsc.VectorSubcoreMesh(core_axis_name="c", subcore_axis_name="s"),
        out_type=jax.ShapeDtypeStruct((BATCH, DIM), jnp.float32),
        scratch_types=[
            pltpu.VMEM((NCHUNK, CHUNK_ROWS), jnp.int32),
            pltpu.VMEM((NCHUNK, SC_ROWS), jnp.int32),
            pltpu.VMEM((NBUF, CHUNK_ROWS, DIM), jnp.float32),
            pltpu.VMEM((SENT_PER_W, DIM), jnp.float32),
            pltpu.VMEM((SENT_PER_W, DIM), jnp.float32),
            pltpu.VMEM_SHARED((SENT_PER_SC, DIM), jnp.float32),
            pltpu.SemaphoreType.DMA((NBUF,)),
            pltpu.SemaphoreType.DMA((NBUF,)),
        ],
    )(_embed_body)


def kernel(indices, table):
    idx = indices.astype(jnp.int32).reshape(NCHUNK_ALL, CHUNK_SENT, SEQ)
    sc_part = idx[:, :, :K_SC].reshape(NCHUNK_ALL, SC_ROWS)
    v_part = idx[:, :, K_SC:].reshape(NCHUNK_ALL, CHUNK_SENT * K_V)
    idx2 = jnp.concatenate([sc_part, v_part], axis=1)
    seg = jnp.asarray(_SEG)
    return _make()(idx2, seg, table)


# speedup vs baseline: 1.3601x; 1.3601x over previous
import functools

import jax
import jax.numpy as jnp
from jax import lax
from jax.experimental import pallas as pl
from jax.experimental.pallas import tpu as pltpu
from jax.experimental.pallas import tpu_sc as plsc

BATCH = 4096
SEQ = 50
DIM = 128
LANES = 16
NCORE = 2
NSUB = 16
SENT_PER_W = BATCH // (NCORE * NSUB)
CHUNK_SENT = 2
CHUNK_ROWS = CHUNK_SENT * SEQ
NCHUNK = SENT_PER_W // CHUNK_SENT
NCHUNK_ALL = BATCH // CHUNK_SENT
LGROUPS = DIM // LANES
NBUF = 4


def _embed_body(idx_hbm, table_hbm, out_hbm,
                idx_v, rows_v, out_v, gsems):
    c = lax.axis_index("c")
    s = lax.axis_index("s")
    sent_base = pl.multiple_of(c * (NSUB * SENT_PER_W) + s * SENT_PER_W,
                               SENT_PER_W)
    chunk_base = pl.multiple_of(sent_base // CHUNK_SENT, NCHUNK)

    pltpu.sync_copy(idx_hbm.at[pl.ds(chunk_base, NCHUNK)], idx_v)

    scale = jnp.float32(1.0 / SEQ)

    def gather(p, buf):
        return pltpu.async_copy(
            table_hbm.at[idx_v.at[p]], rows_v.at[buf], gsems.at[buf])

    def gather_wait(p, buf):
        pltpu.make_async_copy(
            table_hbm.at[idx_v.at[p]], rows_v.at[buf], gsems.at[buf]).wait()

    def reduce_sentence(buf, base_row):
        def body(t, accs):
            r0 = base_row + 2 * t
            return tuple(
                accs[l] + rows_v[buf, r0, pl.ds(l * LANES, LANES)]
                + rows_v[buf, r0 + 1, pl.ds(l * LANES, LANES)]
                for l in range(LGROUPS)
            )
        init = tuple(jnp.zeros((LANES,), jnp.float32) for _ in range(LGROUPS))
        return lax.fori_loop(0, SEQ // 2, body, init)

    def compute(p, buf):
        for s2 in range(CHUNK_SENT):
            accs = reduce_sentence(buf, s2 * SEQ)
            for l in range(LGROUPS):
                out_v[CHUNK_SENT * p + s2, pl.ds(l * LANES, LANES)] = (
                    accs[l] * scale)

    for j in range(NBUF):
        gather(j, j)

    def outer(q, carry):
        for j in range(NBUF):
            p = NBUF * q + j
            gather_wait(p, j)
            compute(p, j)

            @pl.when(q < NCHUNK // NBUF - 1)
            def _():
                gather(p + NBUF, j)
        return carry

    lax.fori_loop(0, NCHUNK // NBUF, outer, 0)

    pltpu.sync_copy(out_v, out_hbm.at[pl.ds(sent_base, SENT_PER_W)])


def _make():
    return functools.partial(
        pl.kernel,
        mesh=plsc.VectorSubcoreMesh(core_axis_name="c", subcore_axis_name="s"),
        out_type=jax.ShapeDtypeStruct((BATCH, DIM), jnp.float32),
        scratch_types=[
            pltpu.VMEM((NCHUNK, CHUNK_ROWS), jnp.int32),
            pltpu.VMEM((NBUF, CHUNK_ROWS, DIM), jnp.float32),
            pltpu.VMEM((SENT_PER_W, DIM), jnp.float32),
            pltpu.SemaphoreType.DMA((NBUF,)),
        ],
    )(_embed_body)


def kernel(indices, table):
    idx = indices.astype(jnp.int32).reshape(NCHUNK_ALL, CHUNK_ROWS)
    return _make()(idx, table)


# speedup vs baseline: 1.3744x; 1.0105x over previous
import functools

import jax
import jax.numpy as jnp
from jax import lax
from jax.experimental import pallas as pl
from jax.experimental.pallas import tpu as pltpu
from jax.experimental.pallas import tpu_sc as plsc

BATCH = 4096
SEQ = 50
DIM = 128
LANES = 16
NCORE = 2
NSUB = 16
NWORKER = NCORE * NSUB
SENT_PER_W = BATCH // NWORKER
LGROUPS = DIM // LANES


def _embed_body(idx_hbm, table_hbm, out_hbm, idx_v, acc_v, gsems):
    c = lax.axis_index("c")
    s = lax.axis_index("s")
    w = c * NSUB + s
    sent_base = pl.multiple_of(w * SENT_PER_W, SENT_PER_W)

    pltpu.sync_copy(idx_hbm.at[w], idx_v)

    zero = jnp.zeros((LANES,), jnp.float32)

    def zbody(r, carry):
        for l in range(LGROUPS):
            acc_v[r, pl.ds(l * LANES, LANES)] = zero
        return carry

    lax.fori_loop(0, SENT_PER_W, zbody, 0)

    for g in range(SEQ):
        pltpu.async_copy(table_hbm.at[idx_v.at[g]], acc_v, gsems.at[0],
                         add=True)
    for g in range(SEQ):
        pltpu.make_async_copy(table_hbm.at[idx_v.at[g]], acc_v,
                              gsems.at[0]).wait()

    scale = jnp.float32(1.0 / SEQ)

    def cbody(r, carry):
        for l in range(LGROUPS):
            sl = pl.ds(l * LANES, LANES)
            acc_v[r, sl] = acc_v[r, sl] * scale
        return carry

    lax.fori_loop(0, SENT_PER_W, cbody, 0)

    pltpu.sync_copy(acc_v, out_hbm.at[pl.ds(sent_base, SENT_PER_W)])


def _make():
    return functools.partial(
        pl.kernel,
        mesh=plsc.VectorSubcoreMesh(core_axis_name="c", subcore_axis_name="s"),
        out_type=jax.ShapeDtypeStruct((BATCH, DIM), jnp.float32),
        scratch_types=[
            pltpu.VMEM((SEQ, SENT_PER_W), jnp.int32),
            pltpu.VMEM((SENT_PER_W, DIM), jnp.float32),
            pltpu.SemaphoreType.DMA((1,)),
        ],
    )(_embed_body)


def kernel(indices, table):
    idx = (indices.astype(jnp.int32)
           .reshape(NWORKER, SENT_PER_W, SEQ)
           .transpose(0, 2, 1))
    return _make()(idx, table)


# speedup vs baseline: 1.3792x; 1.0035x over previous
import functools

import jax
import jax.numpy as jnp
from jax import lax
from jax.experimental import pallas as pl
from jax.experimental.pallas import tpu as pltpu
from jax.experimental.pallas import tpu_sc as plsc

BATCH = 4096
SEQ = 50
DIM = 128
LANES = 16
NCORE = 2
NSUB = 16
NWORKER = NCORE * NSUB
SENT_PER_W = BATCH // NWORKER
LGROUPS = DIM // LANES


def _embed_body(idx_hbm, table_hbm, out_hbm, idx_v, acc_v, gsems):
    c = lax.axis_index("c")
    s = lax.axis_index("s")
    w = c * NSUB + s
    sent_base = pl.multiple_of(w * SENT_PER_W, SENT_PER_W)

    pltpu.sync_copy(idx_hbm.at[w], idx_v)

    zero = jnp.zeros((LANES,), jnp.float32)

    def zbody(r, carry):
        for l in range(LGROUPS):
            acc_v[r, pl.ds(l * LANES, LANES)] = zero
        return carry

    lax.fori_loop(0, SENT_PER_W, zbody, 0)

    def gbody(g, carry):
        pltpu.async_copy(table_hbm.at[idx_v.at[g]], acc_v, gsems.at[0],
                         add=True)
        return carry

    lax.fori_loop(0, SEQ, gbody, 0)

    def wbody(g, carry):
        pltpu.make_async_copy(table_hbm.at[idx_v.at[g]], acc_v,
                              gsems.at[0]).wait()
        return carry

    lax.fori_loop(0, SEQ, wbody, 0)

    scale = jnp.float32(1.0 / SEQ)

    def cbody(r, carry):
        for l in range(LGROUPS):
            sl = pl.ds(l * LANES, LANES)
            acc_v[r, sl] = acc_v[r, sl] * scale
        return carry

    lax.fori_loop(0, SENT_PER_W, cbody, 0)

    pltpu.sync_copy(acc_v, out_hbm.at[pl.ds(sent_base, SENT_PER_W)])


def _make():
    return functools.partial(
        pl.kernel,
        mesh=plsc.VectorSubcoreMesh(core_axis_name="c", subcore_axis_name="s"),
        out_type=jax.ShapeDtypeStruct((BATCH, DIM), jnp.float32),
        scratch_types=[
            pltpu.VMEM((SEQ, SENT_PER_W), jnp.int32),
            pltpu.VMEM((SENT_PER_W, DIM), jnp.float32),
            pltpu.SemaphoreType.DMA((1,)),
        ],
    )(_embed_body)


def kernel(indices, table):
    idx = (indices.astype(jnp.int32)
           .reshape(NWORKER, SENT_PER_W, SEQ)
           .transpose(0, 2, 1))
    return _make()(idx, table)


# speedup vs baseline: 1.3956x; 1.0118x over previous
import functools

import jax
import jax.numpy as jnp
from jax import lax
from jax.experimental import pallas as pl
from jax.experimental.pallas import tpu as pltpu
from jax.experimental.pallas import tpu_sc as plsc

BATCH = 4096
SEQ = 50
DIM = 128
LANES = 16
NCORE = 2
NSUB = 16
NWORKER = NCORE * NSUB
SENT_PER_W = BATCH // NWORKER
LGROUPS = DIM // LANES
NQ = 2


def _embed_body(idx_hbm, table_hbm, out_hbm, idx_v, acc_v, isem, gsems):
    c = lax.axis_index("c")
    s = lax.axis_index("s")
    w = c * NSUB + s
    sent_base = pl.multiple_of(w * SENT_PER_W, SENT_PER_W)

    pltpu.async_copy(idx_hbm.at[w], idx_v, isem)

    zero = jnp.zeros((LANES,), jnp.float32)

    def zbody(r, carry):
        for l in range(LGROUPS):
            acc_v[r, pl.ds(l * LANES, LANES)] = zero
        return carry

    lax.fori_loop(0, SENT_PER_W, zbody, 0)
    pltpu.make_async_copy(idx_hbm.at[w], idx_v, isem).wait()

    def gbody(g, carry):
        pltpu.async_copy(table_hbm.at[idx_v.at[g]], acc_v,
                         gsems.at[lax.rem(g, NQ)], add=True)
        return carry

    lax.fori_loop(0, SEQ, gbody, 0)

    def wbody(g, carry):
        pltpu.make_async_copy(table_hbm.at[idx_v.at[g]], acc_v,
                              gsems.at[lax.rem(g, NQ)]).wait()
        return carry

    lax.fori_loop(0, SEQ, wbody, 0)

    scale = jnp.float32(1.0 / SEQ)

    def cbody(r, carry):
        for l in range(LGROUPS):
            sl = pl.ds(l * LANES, LANES)
            acc_v[r, sl] = acc_v[r, sl] * scale
        return carry

    lax.fori_loop(0, SENT_PER_W, cbody, 0)

    pltpu.sync_copy(acc_v, out_hbm.at[pl.ds(sent_base, SENT_PER_W)])


def _make():
    return functools.partial(
        pl.kernel,
        mesh=plsc.VectorSubcoreMesh(core_axis_name="c", subcore_axis_name="s"),
        out_type=jax.ShapeDtypeStruct((BATCH, DIM), jnp.float32),
        scratch_types=[
            pltpu.VMEM((SEQ, SENT_PER_W), jnp.int32),
            pltpu.VMEM((SENT_PER_W, DIM), jnp.float32),
            pltpu.SemaphoreType.DMA,
            pltpu.SemaphoreType.DMA((NQ,)),
        ],
    )(_embed_body)


def kernel(indices, table):
    idx = (indices.astype(jnp.int32)
           .reshape(NWORKER, SENT_PER_W, SEQ)
           .transpose(0, 2, 1))
    return _make()(idx, table)
